# baseline (speedup 1.0000x reference)
import functools

import jax
import jax.numpy as jnp
from jax import lax
from jax.experimental import pallas as pl
from jax.experimental.pallas import tpu as pltpu
from jax.experimental.pallas import tpu_sc as plsc

NUM_EMB = 1000000
D = 64
QW = NUM_EMB // 4
T = 4096 * 50
NW = 32
C = 128
NBLK = T // (NW * C)
IBLK = 4096 // C


def _body(x_ref, q_ref, s0_ref, s1_ref, z_ref, out_ref,
          idx_v, shl_v, idxw, qw, zv, sv0, sv1, zf0, zf1, deq, semq, sems):
    nc = 2
    wid = lax.axis_index("s") * nc + lax.axis_index("c")

    @pl.loop(0, NBLK)
    def block_body(k):
        b = wid * NBLK + k
        j = b // IBLK
        i0 = (b % IBLK) * C
        pltpu.sync_copy(x_ref.at[pl.ds(j * 4096 + i0, C)], idx_v)

        @pl.loop(0, C // 16)
        def pre_body(g):
            iv = idx_v[pl.ds(g * 16, 16)]
            idxw[0, pl.ds(g * 16, 16)] = iv >> 2
            shl_v[pl.ds(g * 16, 16)] = (3 - (iv & 3)) * 8

        @pl.loop(1, D)
        def colidx_body(c):
            off = c * QW
            for g in range(C // 16):
                sl = pl.ds(g * 16, 16)
                idxw[c, sl] = idxw[0, sl] + off

        cpz = pltpu.async_copy(z_ref.at[idx_v], zv, sems)
        cps0 = pltpu.async_copy(s0_ref.at[idx_v], sv0, sems)
        cps1 = pltpu.async_copy(s1_ref.at[idx_v], sv1, sems)

        @pl.loop(0, D)
        def gather_body(c):
            pltpu.async_copy(q_ref.at[idxw.at[c]], qw.at[pl.ds(c * C, C)],
                             semq)

        cpz.wait()
        cps0.wait()
        cps1.wait()

        @pl.loop(0, C // 16)
        def zpre_body(g):
            zw = zv[pl.ds(g * 16, 16)]
            zf0[pl.ds(g * 16, 16)] = ((zw << 24) >> 24).astype(jnp.float32)
            zf1[pl.ds(g * 16, 16)] = ((zw << 16) >> 24).astype(jnp.float32)

        pltpu.make_async_copy(q_ref.at[pl.ds(0, D * C)], qw, semq).wait()

        @pl.loop(0, C // 16)
        def grp_body(g):
            sl = pl.ds(g * 16, 16)
            shl16 = shl_v[sl]
            s0_16 = sv0[sl]
            s1_16 = sv1[sl]
            z0_16 = zf0[sl]
            z1_16 = zf1[sl]

            @pl.loop(0, D // 2)
            def c_body0(c):
                w = qw[pl.ds(c * C + g * 16, 16)]
                v = ((w << shl16) >> 24).astype(jnp.float32)
                deq[c, sl] = (v - z0_16) * s0_16

            @pl.loop(D // 2, D)
            def c_body1(c):
                w = qw[pl.ds(c * C + g * 16, 16)]
                v = ((w << shl16) >> 24).astype(jnp.float32)
                deq[c, sl] = (v - z1_16) * s1_16

        pltpu.sync_copy(deq, out_ref.at[j, :, pl.ds(i0, C)])


_sc_call = functools.partial(
    pl.kernel,
    out_type=jax.ShapeDtypeStruct((50, D, 4096), jnp.float32),
    mesh=plsc.VectorSubcoreMesh(core_axis_name="c", subcore_axis_name="s"),
    compiler_params=pltpu.CompilerParams(
        needs_layout_passes=False, use_tc_tiling_on_sc=False),
    scratch_types=[
        pltpu.VMEM((C,), jnp.int32),
        pltpu.VMEM((C,), jnp.int32),
        pltpu.VMEM((D, C), jnp.int32),
        pltpu.VMEM((D * C,), jnp.int32),
        pltpu.VMEM((C,), jnp.int32),
        pltpu.VMEM((C,), jnp.float32),
        pltpu.VMEM((C,), jnp.float32),
        pltpu.VMEM((C,), jnp.float32),
        pltpu.VMEM((C,), jnp.float32),
        pltpu.VMEM((D, C), jnp.float32),
        pltpu.SemaphoreType.DMA,
        pltpu.SemaphoreType.DMA,
    ],
)(_body)


@jax.jit
def kernel(x, qvals, scales, zeros):
    xf = x.T.reshape(-1)
    qtab = lax.bitcast_convert_type(
        qvals.T.reshape(D, QW, 4), jnp.int32).reshape(-1)
    stab0 = scales.T[0]
    stab1 = scales.T[1]
    z32 = lax.bitcast_convert_type(zeros, jnp.int16).astype(jnp.int32)
    out3 = _sc_call(xf, qtab, stab0, stab1, z32)
    return out3.transpose(2, 0, 1)
